# baseline (speedup 1.0000x reference)
import functools

import jax
import jax.numpy as jnp
from jax import lax
from jax.experimental import pallas as pl
from jax.experimental.pallas import tpu as pltpu
from jax.experimental.pallas import tpu_sc as plsc

N = 10000
NP = 10240
E = 320000
D = 128
NC = 2
NS = 16
NW = NC * NS
BK = 80
NB = 128
EPAD = NW * NB * BK
ROWS_PER_SUB = NP // NS

_MESH = plsc.VectorSubcoreMesh(core_axis_name="c", subcore_axis_name="s")


def _deg_body(dst3_hbm, ones_hbm, zeros_hbm, out_hbm, didx_v, stage_v, acc_sh):
    c = lax.axis_index("c")
    s = lax.axis_index("s")
    wid = s * NC + c
    pltpu.sync_copy(dst3_hbm.at[wid], didx_v)
    pltpu.sync_copy(ones_hbm, stage_v)
    pltpu.sync_copy(zeros_hbm, acc_sh.at[pl.ds(s * ROWS_PER_SUB, ROWS_PER_SUB)])
    plsc.subcore_barrier()

    def body(i, carry):
        pltpu.sync_copy(stage_v, acc_sh.at[didx_v.at[i]], add=True)
        return carry

    lax.fori_loop(0, NB, body, 0)
    plsc.subcore_barrier()
    pltpu.sync_copy(
        acc_sh.at[pl.ds(s * ROWS_PER_SUB, ROWS_PER_SUB)],
        out_hbm.at[c, pl.ds(s * ROWS_PER_SUB, ROWS_PER_SUB)],
    )


_deg_kernel = functools.partial(
    pl.kernel,
    out_type=jax.ShapeDtypeStruct((NC, NP, D), jnp.float32),
    mesh=_MESH,
    scratch_types=[
        pltpu.VMEM((NB, BK), jnp.int32),
        pltpu.VMEM((BK, D), jnp.float32),
        pltpu.VMEM_SHARED((NP, D), jnp.float32),
    ],
)(_deg_body)


NBUF = 4
NCHUNK = NB // NBUF
GA = 3


def _segsum_body(y_hbm, src3_hbm, dst3_hbm, zeros_hbm, out_hbm,
                 sidx, didx, stages, isem, gsem, acc_sh):
    c = lax.axis_index("c")
    s = lax.axis_index("s")
    wid = s * NC + c

    def fire_idx(i, t):
        pltpu.async_copy(src3_hbm.at[wid, i], sidx[t], isem[t])
        pltpu.async_copy(dst3_hbm.at[wid, i], didx[t].at[0], isem[t])

    def wait_idx(i, t):
        pltpu.make_async_copy(src3_hbm.at[wid, i], sidx[t], isem[t]).wait()
        pltpu.make_async_copy(dst3_hbm.at[wid, i], didx[t].at[0], isem[t]).wait()

    def fire_gather(i, t):
        pltpu.async_copy(y_hbm.at[sidx[t]], stages[t], gsem[t])

    for t in range(NBUF):
        fire_idx(t, t)
    for t in range(GA):
        wait_idx(t, t)
        fire_gather(t, t)
    pltpu.sync_copy(zeros_hbm, acc_sh.at[pl.ds(s * ROWS_PER_SUB, ROWS_PER_SUB)])
    plsc.subcore_barrier()

    def chunk(j, carry):
        for t in range(NBUF):
            i = j * NBUF + t
            pltpu.make_async_copy(y_hbm.at[sidx[t]], stages[t], gsem[t]).wait()
            pltpu.sync_copy(stages[t], acc_sh.at[didx[t].at[0]], add=True)

            @pl.when(i + NBUF < NB)
            def _():
                fire_idx(i + NBUF, t)

            u = (t + GA) % NBUF

            @pl.when(i + GA < NB)
            def _():
                wait_idx(i + GA, u)
                fire_gather(i + GA, u)
        return carry

    lax.fori_loop(0, NCHUNK, chunk, 0)
    plsc.subcore_barrier()
    pltpu.sync_copy(
        acc_sh.at[pl.ds(s * ROWS_PER_SUB, ROWS_PER_SUB)],
        out_hbm.at[c, pl.ds(s * ROWS_PER_SUB, ROWS_PER_SUB)],
    )


_segsum_kernel = functools.partial(
    pl.kernel,
    out_type=jax.ShapeDtypeStruct((NC, NP, D), jnp.float32),
    mesh=_MESH,
    scratch_types={
        "sidx": [pltpu.VMEM((BK,), jnp.int32)] * NBUF,
        "didx": [pltpu.VMEM((1, BK), jnp.int32)] * NBUF,
        "stages": [pltpu.VMEM((BK, D), jnp.float32)] * NBUF,
        "isem": [pltpu.SemaphoreType.DMA] * NBUF,
        "gsem": [pltpu.SemaphoreType.DMA] * NBUF,
        "acc_sh": pltpu.VMEM_SHARED((NP, D), jnp.float32),
    },
)(_segsum_body)


_RB = 1000
_GRID = (N // _RB,)


def _dinv_block(degp_ref):
    deg = 1.0 + degp_ref[0, :, 0] + degp_ref[1, :, 0]
    return lax.rsqrt(deg)[:, None]


def _tc1_body(x_ref, w_ref, degp_ref, y_ref):
    dinv = _dinv_block(degp_ref)
    y_ref[...] = jnp.dot(x_ref[...], w_ref[...],
                         preferred_element_type=jnp.float32) * dinv


def _tc2_body(s_ref, y1_ref, degp_ref, w_ref, b_ref, y2_ref):
    dinv = _dinv_block(degp_ref)
    h = jnp.maximum((s_ref[0] + s_ref[1] + y1_ref[...]) * dinv + b_ref[...], 0.0)
    y2_ref[...] = jnp.dot(h, w_ref[...],
                          preferred_element_type=jnp.float32) * dinv


def _tc3_body(s_ref, y2_ref, degp_ref, b_ref, o_ref):
    dinv = _dinv_block(degp_ref)
    o_ref[...] = (s_ref[0] + s_ref[1] + y2_ref[...]) * dinv + b_ref[...]


_row_spec = pl.BlockSpec((_RB, D), lambda i: (i, 0))
_w_spec = pl.BlockSpec((D, D), lambda i: (0, 0))
_part_spec = pl.BlockSpec((NC, _RB, D), lambda i: (0, i, 0))
_degp_spec = _part_spec
_b_spec = pl.BlockSpec((1, D), lambda i: (0, 0))

_tc1 = pl.pallas_call(
    _tc1_body,
    grid=_GRID,
    in_specs=[_row_spec, _w_spec, _degp_spec],
    out_specs=_row_spec,
    out_shape=jax.ShapeDtypeStruct((N, D), jnp.float32),
)

_tc2 = pl.pallas_call(
    _tc2_body,
    grid=_GRID,
    in_specs=[_part_spec, _row_spec, _degp_spec, _w_spec, _b_spec],
    out_specs=_row_spec,
    out_shape=jax.ShapeDtypeStruct((N, D), jnp.float32),
)

_tc3 = pl.pallas_call(
    _tc3_body,
    grid=_GRID,
    in_specs=[_part_spec, _row_spec, _degp_spec, _b_spec],
    out_specs=_row_spec,
    out_shape=jax.ShapeDtypeStruct((N, D), jnp.float32),
)


def kernel(x, edge_index, W_shared, b_shared, W_mu, b_mu, W_logvar, b_logvar):
    src = edge_index[0].astype(jnp.int32)
    dst = edge_index[1].astype(jnp.int32)
    pad = EPAD - E
    src3 = jnp.concatenate([src, jnp.zeros((pad,), jnp.int32)]).reshape(NW, NB, BK)
    dst3 = jnp.concatenate([dst, jnp.full((pad,), NP - 1, jnp.int32)]).reshape(NW, NB, BK)
    zeros_rows = jnp.zeros((ROWS_PER_SUB, D), jnp.float32)
    ones_blk = jnp.ones((BK, D), jnp.float32)

    degp = _deg_kernel(dst3, ones_blk, zeros_rows)
    y1 = _tc1(x, W_shared, degp)
    s1p = _segsum_kernel(y1, src3, dst3, zeros_rows)
    w_cat = jnp.concatenate([W_mu, W_logvar], axis=1)
    b_cat = jnp.concatenate([b_mu, b_logvar])[None, :]
    y2 = _tc2(s1p, y1, degp, w_cat, b_shared[None, :])
    s2p = _segsum_kernel(y2, src3, dst3, zeros_rows)
    out = _tc3(s2p, y2, degp, b_cat)
    return out[:, :64], out[:, 64:]


# speedup vs baseline: 3.0895x; 3.0895x over previous
import functools

import jax
import jax.numpy as jnp
from jax import lax
from jax.experimental import pallas as pl
from jax.experimental.pallas import tpu as pltpu
from jax.experimental.pallas import tpu_sc as plsc

N = 10000
NP = 10240
E = 320000
D = 128
NC = 2
NS = 16
NW = NC * NS
BK = 80
NB = 128
EPAD = NW * NB * BK
ROWS_PER_SUB = NP // NS

_MESH = plsc.VectorSubcoreMesh(core_axis_name="c", subcore_axis_name="s")


def _deg_body(dst3_hbm, ones_hbm, zeros_hbm, out_hbm, didx_v, stage_v, acc_sh):
    c = lax.axis_index("c")
    s = lax.axis_index("s")
    wid = s * NC + c
    pltpu.sync_copy(dst3_hbm.at[wid], didx_v)
    pltpu.sync_copy(ones_hbm, stage_v)
    pltpu.sync_copy(zeros_hbm, acc_sh.at[pl.ds(s * ROWS_PER_SUB, ROWS_PER_SUB)])
    plsc.subcore_barrier()

    def body(i, carry):
        pltpu.sync_copy(stage_v, acc_sh.at[didx_v.at[i]], add=True)
        return carry

    lax.fori_loop(0, NB, body, 0)
    plsc.subcore_barrier()
    pltpu.sync_copy(
        acc_sh.at[pl.ds(s * ROWS_PER_SUB, ROWS_PER_SUB)],
        out_hbm.at[c, pl.ds(s * ROWS_PER_SUB, ROWS_PER_SUB)],
    )


_deg_kernel = functools.partial(
    pl.kernel,
    out_type=jax.ShapeDtypeStruct((NC, NP, D), jnp.float32),
    mesh=_MESH,
    scratch_types=[
        pltpu.VMEM((NB, BK), jnp.int32),
        pltpu.VMEM((BK, D), jnp.float32),
        pltpu.VMEM_SHARED((NP, D), jnp.float32),
    ],
)(_deg_body)


NBUF = 4
NCHUNK = NB // NBUF
GA = 3


def _segsum_body(y_hbm, src3_hbm, dst3_hbm, zeros_hbm, out_hbm,
                 sidx, didx, stages, isem, gsem, acc_sh):
    c = lax.axis_index("c")
    s = lax.axis_index("s")
    wid = s * NC + c

    def fire_idx(i, t):
        pltpu.async_copy(src3_hbm.at[wid, i], sidx[t], isem[t])
        pltpu.async_copy(dst3_hbm.at[wid, i], didx[t].at[0], isem[t])

    def wait_idx(i, t):
        pltpu.make_async_copy(src3_hbm.at[wid, i], sidx[t], isem[t]).wait()
        pltpu.make_async_copy(dst3_hbm.at[wid, i], didx[t].at[0], isem[t]).wait()

    def fire_gather(i, t):
        pltpu.async_copy(y_hbm.at[sidx[t]], stages[t], gsem[t])

    for t in range(NBUF):
        fire_idx(t, t)
    for t in range(GA):
        wait_idx(t, t)
        fire_gather(t, t)
    pltpu.sync_copy(zeros_hbm, acc_sh.at[pl.ds(s * ROWS_PER_SUB, ROWS_PER_SUB)])
    plsc.subcore_barrier()

    def chunk(j, carry):
        for t in range(NBUF):
            i = j * NBUF + t
            pltpu.make_async_copy(y_hbm.at[sidx[t]], stages[t], gsem[t]).wait()
            pltpu.sync_copy(stages[t], acc_sh.at[didx[t].at[0]], add=True)

            @pl.when(i + NBUF < NB)
            def _():
                fire_idx(i + NBUF, t)

            u = (t + GA) % NBUF

            @pl.when(i + GA < NB)
            def _():
                wait_idx(i + GA, u)
                fire_gather(i + GA, u)
        return carry

    lax.fori_loop(0, NCHUNK, chunk, 0)
    plsc.subcore_barrier()
    pltpu.sync_copy(
        acc_sh.at[pl.ds(s * ROWS_PER_SUB, ROWS_PER_SUB)],
        out_hbm.at[c, pl.ds(s * ROWS_PER_SUB, ROWS_PER_SUB)],
    )


_segsum_kernel = functools.partial(
    pl.kernel,
    out_type=jax.ShapeDtypeStruct((NC, NP, D), jnp.float32),
    mesh=_MESH,
    scratch_types={
        "sidx": [pltpu.VMEM((BK,), jnp.int32)] * NBUF,
        "didx": [pltpu.VMEM((1, BK), jnp.int32)] * NBUF,
        "stages": [pltpu.VMEM((BK, D), jnp.float32)] * NBUF,
        "isem": [pltpu.SemaphoreType.DMA] * NBUF,
        "gsem": [pltpu.SemaphoreType.DMA] * NBUF,
        "acc_sh": pltpu.VMEM_SHARED((NP, D), jnp.float32),
    },
)(_segsum_body)


_RB = 1000
_GRID = (N // _RB,)


def _dinv_block(degp_ref):
    deg = 1.0 + degp_ref[0, :, 0] + degp_ref[1, :, 0]
    return lax.rsqrt(deg)[:, None]


def _tc1_body(x_ref, w_ref, degp_ref, y_ref):
    dinv = _dinv_block(degp_ref)
    y_ref[...] = jnp.dot(x_ref[...], w_ref[...],
                         preferred_element_type=jnp.float32) * dinv


def _tc2_body(s_ref, y1_ref, degp_ref, w_ref, b_ref, y2_ref):
    dinv = _dinv_block(degp_ref)
    h = jnp.maximum((s_ref[0] + s_ref[1] + y1_ref[...]) * dinv + b_ref[...], 0.0)
    y2_ref[...] = jnp.dot(h, w_ref[...],
                          preferred_element_type=jnp.float32) * dinv


def _tc3_body(s_ref, y2_ref, degp_ref, b_ref, o_ref):
    dinv = _dinv_block(degp_ref)
    o_ref[...] = (s_ref[0] + s_ref[1] + y2_ref[...]) * dinv + b_ref[...]


_row_spec = pl.BlockSpec((_RB, D), lambda i: (i, 0))
_w_spec = pl.BlockSpec((D, D), lambda i: (0, 0))
_part_spec = pl.BlockSpec((NC, _RB, D), lambda i: (0, i, 0))
_degp_spec = _part_spec
_b_spec = pl.BlockSpec((1, D), lambda i: (0, 0))

_tc1 = pl.pallas_call(
    _tc1_body,
    grid=_GRID,
    in_specs=[_row_spec, _w_spec, _degp_spec],
    out_specs=_row_spec,
    out_shape=jax.ShapeDtypeStruct((N, D), jnp.float32),
)

_tc2 = pl.pallas_call(
    _tc2_body,
    grid=_GRID,
    in_specs=[_part_spec, _row_spec, _degp_spec, _w_spec, _b_spec],
    out_specs=_row_spec,
    out_shape=jax.ShapeDtypeStruct((N, D), jnp.float32),
)

_tc3 = pl.pallas_call(
    _tc3_body,
    grid=_GRID,
    in_specs=[_part_spec, _row_spec, _degp_spec, _b_spec],
    out_specs=_row_spec,
    out_shape=jax.ShapeDtypeStruct((N, D), jnp.float32),
)


def kernel(x, edge_index, W_shared, b_shared, W_mu, b_mu, W_logvar, b_logvar):
    src = edge_index[0].astype(jnp.int32)
    dst = edge_index[1].astype(jnp.int32)
    pad = EPAD - E
    pad_src = (jnp.arange(pad, dtype=jnp.int32) * 131) % N
    pad_dst = N + (jnp.arange(pad, dtype=jnp.int32) % (NP - N))
    src3 = jnp.concatenate([src, pad_src]).reshape(NW, NB, BK)
    dst3 = jnp.concatenate([dst, pad_dst]).reshape(NW, NB, BK)
    zeros_rows = jnp.zeros((ROWS_PER_SUB, D), jnp.float32)
    ones_blk = jnp.ones((BK, D), jnp.float32)

    degp = _deg_kernel(dst3, ones_blk, zeros_rows)
    y1 = _tc1(x, W_shared, degp)
    s1p = _segsum_kernel(y1, src3, dst3, zeros_rows)
    w_cat = jnp.concatenate([W_mu, W_logvar], axis=1)
    b_cat = jnp.concatenate([b_mu, b_logvar])[None, :]
    y2 = _tc2(s1p, y1, degp, w_cat, b_shared[None, :])
    s2p = _segsum_kernel(y2, src3, dst3, zeros_rows)
    out = _tc3(s2p, y2, degp, b_cat)
    return out[:, :64], out[:, 64:]


# speedup vs baseline: 3.7882x; 1.2262x over previous
import functools

import jax
import jax.numpy as jnp
from jax import lax
from jax.experimental import pallas as pl
from jax.experimental.pallas import tpu as pltpu
from jax.experimental.pallas import tpu_sc as plsc

N = 10000
NP = 10240
E = 320000
D = 128
NC = 2
NS = 16
NW = NC * NS
BK = 80
NB = 128
EPAD = NW * NB * BK
ROWS_PER_SUB = NP // NS

_MESH = plsc.VectorSubcoreMesh(core_axis_name="c", subcore_axis_name="s")


EPT = NB * BK
L = 16


def _deg_body(dstf_hbm, out_hbm, didx_v, hist_v, tmp_v, red_v, hist_sh):
    c = lax.axis_index("c")
    s = lax.axis_index("s")
    wid = s * NC + c
    pltpu.sync_copy(dstf_hbm.at[wid], didx_v)

    def zero(i, carry):
        hist_v[pl.ds(i * L, L)] = jnp.zeros((L,), jnp.float32)
        return carry

    lax.fori_loop(0, NP // L, zero, 0)
    ones = jnp.ones((L,), jnp.float32)

    def step(i, carry):
        plsc.addupdate_scatter(hist_v, [didx_v[pl.ds(i * L, L)]], ones)
        return carry

    lax.fori_loop(0, EPT // L, step, 0)
    pltpu.sync_copy(hist_v, hist_sh.at[s])
    plsc.subcore_barrier()

    def zred(i, carry):
        red_v[pl.ds(i * L, L)] = jnp.zeros((L,), jnp.float32)
        return carry

    lax.fori_loop(0, ROWS_PER_SUB // L, zred, 0)
    for t in range(NS):
        pltpu.sync_copy(hist_sh.at[t, pl.ds(s * ROWS_PER_SUB, ROWS_PER_SUB)], tmp_v)

        def add(i, carry):
            sl = pl.ds(i * L, L)
            red_v[sl] = red_v[sl] + tmp_v[sl]
            return carry

        lax.fori_loop(0, ROWS_PER_SUB // L, add, 0)
    pltpu.sync_copy(red_v, out_hbm.at[c, pl.ds(s * ROWS_PER_SUB, ROWS_PER_SUB)])


_deg_kernel = functools.partial(
    pl.kernel,
    out_type=jax.ShapeDtypeStruct((NC, NP), jnp.float32),
    mesh=_MESH,
    scratch_types=[
        pltpu.VMEM((EPT,), jnp.int32),
        pltpu.VMEM((NP,), jnp.float32),
        pltpu.VMEM((ROWS_PER_SUB,), jnp.float32),
        pltpu.VMEM((ROWS_PER_SUB,), jnp.float32),
        pltpu.VMEM_SHARED((NS, NP), jnp.float32),
    ],
    compiler_params=pltpu.CompilerParams(needs_layout_passes=False),
)(_deg_body)


NBUF = 4
NCHUNK = NB // NBUF
GA = 3


def _segsum_body(y_hbm, src3_hbm, dst3_hbm, zeros_hbm, out_hbm,
                 sidx, didx, stages, isem, gsem, acc_sh):
    c = lax.axis_index("c")
    s = lax.axis_index("s")
    wid = s * NC + c

    def fire_idx(i, t):
        pltpu.async_copy(src3_hbm.at[wid, i], sidx[t], isem[t])
        pltpu.async_copy(dst3_hbm.at[wid, i], didx[t].at[0], isem[t])

    def wait_idx(i, t):
        pltpu.make_async_copy(src3_hbm.at[wid, i], sidx[t], isem[t]).wait()
        pltpu.make_async_copy(dst3_hbm.at[wid, i], didx[t].at[0], isem[t]).wait()

    def fire_gather(i, t):
        pltpu.async_copy(y_hbm.at[sidx[t]], stages[t], gsem[t])

    for t in range(NBUF):
        fire_idx(t, t)
    for t in range(GA):
        wait_idx(t, t)
        fire_gather(t, t)
    pltpu.sync_copy(zeros_hbm, acc_sh.at[pl.ds(s * ROWS_PER_SUB, ROWS_PER_SUB)])
    plsc.subcore_barrier()

    def chunk(j, carry):
        for t in range(NBUF):
            i = j * NBUF + t
            pltpu.make_async_copy(y_hbm.at[sidx[t]], stages[t], gsem[t]).wait()
            pltpu.sync_copy(stages[t], acc_sh.at[didx[t].at[0]], add=True)

            @pl.when(i + NBUF < NB)
            def _():
                fire_idx(i + NBUF, t)

            u = (t + GA) % NBUF

            @pl.when(i + GA < NB)
            def _():
                wait_idx(i + GA, u)
                fire_gather(i + GA, u)
        return carry

    lax.fori_loop(0, NCHUNK, chunk, 0)
    plsc.subcore_barrier()
    pltpu.sync_copy(
        acc_sh.at[pl.ds(s * ROWS_PER_SUB, ROWS_PER_SUB)],
        out_hbm.at[c, pl.ds(s * ROWS_PER_SUB, ROWS_PER_SUB)],
    )


_segsum_kernel = functools.partial(
    pl.kernel,
    out_type=jax.ShapeDtypeStruct((NC, NP, D), jnp.float32),
    mesh=_MESH,
    scratch_types={
        "sidx": [pltpu.VMEM((BK,), jnp.int32)] * NBUF,
        "didx": [pltpu.VMEM((1, BK), jnp.int32)] * NBUF,
        "stages": [pltpu.VMEM((BK, D), jnp.float32)] * NBUF,
        "isem": [pltpu.SemaphoreType.DMA] * NBUF,
        "gsem": [pltpu.SemaphoreType.DMA] * NBUF,
        "acc_sh": pltpu.VMEM_SHARED((NP, D), jnp.float32),
    },
)(_segsum_body)


_RB = 1024
_GRID = (NP // _RB,)


def _tc_mm_body(x_ref, w_ref, degp_ref, y_ref):
    y_ref[...] = jnp.dot(x_ref[...], w_ref[...],
                         preferred_element_type=jnp.float32) * _dinv_block(degp_ref)


def _dinv_block(degp_ref):
    deg = 1.0 + degp_ref[0] + degp_ref[1]
    return lax.rsqrt(deg)[:, None]


def _tc2_body(s_ref, y1_ref, degp_ref, w_ref, b_ref, y2_ref):
    dinv = _dinv_block(degp_ref)
    h = jnp.maximum((s_ref[0] + s_ref[1] + y1_ref[...]) * dinv + b_ref[...], 0.0)
    y2_ref[...] = jnp.dot(h, w_ref[...],
                          preferred_element_type=jnp.float32) * dinv


def _tc3_body(s_ref, y2_ref, degp_ref, b_ref, o_ref):
    dinv = _dinv_block(degp_ref)
    o_ref[...] = (s_ref[0] + s_ref[1] + y2_ref[...]) * dinv + b_ref[...]


_row_spec = pl.BlockSpec((_RB, D), lambda i: (i, 0))
_w_spec = pl.BlockSpec((D, D), lambda i: (0, 0))
_part_spec = pl.BlockSpec((NC, _RB, D), lambda i: (0, i, 0))
_degp_spec = pl.BlockSpec((NC, _RB), lambda i: (0, i))
_b_spec = pl.BlockSpec((1, D), lambda i: (0, 0))

_tc_mm = pl.pallas_call(
    _tc_mm_body,
    grid=_GRID,
    in_specs=[_row_spec, _w_spec, _degp_spec],
    out_specs=_row_spec,
    out_shape=jax.ShapeDtypeStruct((N, D), jnp.float32),
)

_tc2 = pl.pallas_call(
    _tc2_body,
    grid=_GRID,
    in_specs=[_part_spec, _row_spec, _degp_spec, _w_spec, _b_spec],
    out_specs=_row_spec,
    out_shape=jax.ShapeDtypeStruct((N, D), jnp.float32),
)

_tc3 = pl.pallas_call(
    _tc3_body,
    grid=_GRID,
    in_specs=[_part_spec, _row_spec, _degp_spec, _b_spec],
    out_specs=_row_spec,
    out_shape=jax.ShapeDtypeStruct((N, D), jnp.float32),
)


def kernel(x, edge_index, W_shared, b_shared, W_mu, b_mu, W_logvar, b_logvar):
    src = edge_index[0].astype(jnp.int32)
    dst = edge_index[1].astype(jnp.int32)
    pad = EPAD - E
    pad_src = (jnp.arange(pad, dtype=jnp.int32) * 131) % N
    pad_dst = N + (jnp.arange(pad, dtype=jnp.int32) % (NP - N))
    dst_pad = jnp.concatenate([dst, pad_dst])
    src3 = jnp.concatenate([src, pad_src]).reshape(NW, NB, BK)
    dst3 = dst_pad.reshape(NW, NB, BK)
    dstf = dst_pad.reshape(NW, EPT)
    zeros_rows = jnp.zeros((ROWS_PER_SUB, D), jnp.float32)

    degp = _deg_kernel(dstf)
    y1 = _tc_mm(x, W_shared, degp)
    s1p = _segsum_kernel(y1, src3, dst3, zeros_rows)
    w_cat = jnp.concatenate([W_mu, W_logvar], axis=1)
    b_cat = jnp.concatenate([b_mu, b_logvar])[None, :]
    y2 = _tc2(s1p, y1, degp, w_cat, b_shared[None, :])
    s2p = _segsum_kernel(y2, src3, dst3, zeros_rows)
    out = _tc3(s2p, y2, degp, b_cat)
    return out[:, :64], out[:, 64:]
